# Initial kernel scaffold; baseline (speedup 1.0000x reference)
import jax
import jax.numpy as jnp
from jax.experimental import pallas as pl


def kernel(user_features, item_features, edge_index, Wu, bu, Wi, bi, Wum, bum, Wim, bim):
    raise NotImplementedError("write your pallas kernel here")


# speedup vs baseline: 5.6869x; 5.6869x over previous
import functools

import jax
import jax.numpy as jnp
from jax import lax
from jax.experimental import pallas as pl
from jax.experimental.pallas import tpu as pltpu
from jax.experimental.pallas import tpu_sc as plsc

N_NODES = 10000
D = 128
N_EDGES = 320000
CHUNK = 128
N_CHUNKS = N_EDGES // CHUNK
N_SUBCORES = 16
FULL_ROUNDS = N_CHUNKS // N_SUBCORES
TAIL = N_CHUNKS - FULL_ROUNDS * N_SUBCORES
SLAB_ROWS = 1000
N_SLABS = N_NODES // SLAB_ROWS

ROW_BLK = 2000


def _tc_body(uf, itf, wu, bu, wi, bi, wum, bum, wim, bim,
             uemb, gu, iemb, gi):
    dn = (((1,), (1,)), ((), ()))
    u = uf[...]
    t = itf[...]
    uemb[...] = lax.dot_general(u, wu[...], dn, preferred_element_type=jnp.float32) + bu[...]
    gu[...] = lax.dot_general(u, wum[...], dn, preferred_element_type=jnp.float32) + bum[...]
    iemb[...] = lax.dot_general(t, wi[...], dn, preferred_element_type=jnp.float32) + bi[...]
    gi[...] = lax.dot_general(t, wim[...], dn, preferred_element_type=jnp.float32) + bim[...]


def _tc_stage(uf, itf, Wu, bu, Wi, bi, Wum, bum, Wim, bim):
    blk = pl.BlockSpec((ROW_BLK, D), lambda i: (i, 0))
    full = pl.BlockSpec((D, D), lambda i: (0, 0))
    bias = pl.BlockSpec((1, D), lambda i: (0, 0))
    out_sds = jax.ShapeDtypeStruct((N_NODES, D), jnp.float32)
    return pl.pallas_call(
        _tc_body,
        grid=(N_NODES // ROW_BLK,),
        in_specs=[blk, blk, full, bias, full, bias, full, bias, full, bias],
        out_specs=[blk, blk, blk, blk],
        out_shape=[out_sds, out_sds, out_sds, out_sds],
    )(uf, itf, Wu, bu.reshape(1, D), Wi, bi.reshape(1, D),
      Wum, bum.reshape(1, D), Wim, bim.reshape(1, D))


def _sc_direction(src_g, init_emb, src_idx_hbm, dst_idx_hbm, out_hbm,
                  accum, idx_s, idx_d, rows, sem, s):
    slab = pl.ds(s * SLAB_ROWS, SLAB_ROWS)

    @pl.when(s < N_SLABS)
    def _init():
        pltpu.sync_copy(init_emb.at[slab], accum.at[slab])

    plsc.subcore_barrier()

    def do_chunk(k):
        base = k * CHUNK
        pltpu.sync_copy(src_idx_hbm.at[pl.ds(base, CHUNK)], idx_s)
        pltpu.sync_copy(dst_idx_hbm.at[pl.ds(base, CHUNK)], idx_d)
        pltpu.async_copy(src_g.at[idx_s], rows, sem).wait()
        pltpu.sync_copy(rows, accum.at[idx_d], add=True)

    def body(j, carry):
        do_chunk(s + j * N_SUBCORES)
        return carry

    lax.fori_loop(0, FULL_ROUNDS, body, 0)

    @pl.when(s < TAIL)
    def _tail():
        do_chunk(FULL_ROUNDS * N_SUBCORES + s)

    plsc.subcore_barrier()

    @pl.when(s < N_SLABS)
    def _writeout():
        pltpu.sync_copy(accum.at[slab], out_hbm.at[slab])


def _sc_body(gu, gi, uemb, iemb, uidx, iidx, user_out, item_out,
             accum, idx_s, idx_d, rows, sem):
    c = lax.axis_index("c")
    s = lax.axis_index("s")

    @pl.when(c == 0)
    def _items():
        _sc_direction(gu, iemb, uidx, iidx, item_out,
                      accum, idx_s, idx_d, rows, sem, s)

    @pl.when(c == 1)
    def _users():
        _sc_direction(gi, uemb, iidx, uidx, user_out,
                      accum, idx_s, idx_d, rows, sem, s)


@functools.cache
def _sc_stage():
    return pl.kernel(
        _sc_body,
        out_type=[jax.ShapeDtypeStruct((N_NODES, D), jnp.float32),
                  jax.ShapeDtypeStruct((N_NODES, D), jnp.float32)],
        mesh=plsc.VectorSubcoreMesh(core_axis_name="c", subcore_axis_name="s"),
        scratch_types=[
            pltpu.VMEM_SHARED((N_NODES, D), jnp.float32),
            pltpu.VMEM((CHUNK,), jnp.int32),
            pltpu.VMEM((CHUNK,), jnp.int32),
            pltpu.VMEM((CHUNK, D), jnp.float32),
            pltpu.SemaphoreType.DMA,
        ],
    )


def kernel(user_features, item_features, edge_index, Wu, bu, Wi, bi,
           Wum, bum, Wim, bim):
    uemb, gu, iemb, gi = _tc_stage(user_features, item_features,
                                   Wu, bu, Wi, bi, Wum, bum, Wim, bim)
    u_idx = edge_index[0].astype(jnp.int32)
    i_idx = edge_index[1].astype(jnp.int32)
    user_out, item_out = _sc_stage()(gu, gi, uemb, iemb, u_idx, i_idx)
    return (user_out, item_out)


# speedup vs baseline: 9.0575x; 1.5927x over previous
import functools

import jax
import jax.numpy as jnp
from jax import lax
from jax.experimental import pallas as pl
from jax.experimental.pallas import tpu as pltpu
from jax.experimental.pallas import tpu_sc as plsc

N_NODES = 10000
D = 128
N_EDGES = 320000
CHUNK = 128
N_CHUNKS = N_EDGES // CHUNK
N_SUBCORES = 16
FULL_ROUNDS = N_CHUNKS // N_SUBCORES
TAIL = N_CHUNKS - FULL_ROUNDS * N_SUBCORES
SLAB_ROWS = 1000
N_SLABS = N_NODES // SLAB_ROWS

ROW_BLK = 2000


def _tc_body(uf, itf, wu, bu, wi, bi, wum, bum, wim, bim,
             uemb, gu, iemb, gi):
    dn = (((1,), (1,)), ((), ()))
    u = uf[...]
    t = itf[...]
    uemb[...] = lax.dot_general(u, wu[...], dn, preferred_element_type=jnp.float32) + bu[...]
    gu[...] = lax.dot_general(u, wum[...], dn, preferred_element_type=jnp.float32) + bum[...]
    iemb[...] = lax.dot_general(t, wi[...], dn, preferred_element_type=jnp.float32) + bi[...]
    gi[...] = lax.dot_general(t, wim[...], dn, preferred_element_type=jnp.float32) + bim[...]


def _tc_stage(uf, itf, Wu, bu, Wi, bi, Wum, bum, Wim, bim):
    blk = pl.BlockSpec((ROW_BLK, D), lambda i: (i, 0))
    full = pl.BlockSpec((D, D), lambda i: (0, 0))
    bias = pl.BlockSpec((1, D), lambda i: (0, 0))
    out_sds = jax.ShapeDtypeStruct((N_NODES, D), jnp.float32)
    return pl.pallas_call(
        _tc_body,
        grid=(N_NODES // ROW_BLK,),
        in_specs=[blk, blk, full, bias, full, bias, full, bias, full, bias],
        out_specs=[blk, blk, blk, blk],
        out_shape=[out_sds, out_sds, out_sds, out_sds],
    )(uf, itf, Wu, bu.reshape(1, D), Wi, bi.reshape(1, D),
      Wum, bum.reshape(1, D), Wim, bim.reshape(1, D))


def _sc_direction(src_g, init_emb, src_idx_hbm, dst_idx_hbm, out_hbm,
                  accum, idx_s, idx_d, rows, gsem, ssem, s):
    slab = pl.ds(s * SLAB_ROWS, SLAB_ROWS)

    @pl.when(s < N_SLABS)
    def _init():
        pltpu.sync_copy(init_emb.at[slab], accum.at[slab])

    plsc.subcore_barrier()

    def fetch_and_fire(b, j):
        base = (s + j * N_SUBCORES) * CHUNK
        pltpu.sync_copy(src_idx_hbm.at[pl.ds(base, CHUNK)], idx_s[b])
        pltpu.sync_copy(dst_idx_hbm.at[pl.ds(base, CHUNK)], idx_d[b])
        pltpu.async_copy(src_g.at[idx_s[b]], rows[b], gsem[b])

    def wait_gather(b):
        pltpu.make_async_copy(src_g.at[idx_s[b]], rows[b], gsem[b]).wait()

    def fire_scatter(b):
        pltpu.async_copy(rows[b], accum.at[idx_d[b]], ssem[b], add=True)

    def wait_scatter(b):
        pltpu.make_async_copy(rows[b], accum.at[idx_d[b]], ssem[b]).wait()

    fetch_and_fire(0, 0)

    def pair(m, carry):
        for b in (0, 1):
            j = 2 * m + b
            nb = 1 - b

            @pl.when(j > 0)
            def _drain():
                wait_scatter(nb)

            @pl.when(j + 1 < FULL_ROUNDS)
            def _prefetch():
                fetch_and_fire(nb, j + 1)

            wait_gather(b)
            fire_scatter(b)
        return carry

    lax.fori_loop(0, FULL_ROUNDS // 2, pair, 0)
    wait_scatter(1)

    @pl.when(s < TAIL)
    def _tail():
        fetch_and_fire(0, FULL_ROUNDS)
        wait_gather(0)
        pltpu.sync_copy(rows[0], accum.at[idx_d[0]], add=True)

    plsc.subcore_barrier()

    @pl.when(s < N_SLABS)
    def _writeout():
        pltpu.sync_copy(accum.at[slab], out_hbm.at[slab])


def _sc_body(gu, gi, uemb, iemb, uidx, iidx, user_out, item_out,
             accum, idx_s0, idx_s1, idx_d0, idx_d1, rows0, rows1,
             gsem0, gsem1, ssem0, ssem1):
    c = lax.axis_index("c")
    s = lax.axis_index("s")
    idx_s, idx_d = (idx_s0, idx_s1), (idx_d0, idx_d1)
    rows, gsem, ssem = (rows0, rows1), (gsem0, gsem1), (ssem0, ssem1)

    @pl.when(c == 0)
    def _items():
        _sc_direction(gu, iemb, uidx, iidx, item_out,
                      accum, idx_s, idx_d, rows, gsem, ssem, s)

    @pl.when(c == 1)
    def _users():
        _sc_direction(gi, uemb, iidx, uidx, user_out,
                      accum, idx_s, idx_d, rows, gsem, ssem, s)


@functools.cache
def _sc_stage():
    return pl.kernel(
        _sc_body,
        out_type=[jax.ShapeDtypeStruct((N_NODES, D), jnp.float32),
                  jax.ShapeDtypeStruct((N_NODES, D), jnp.float32)],
        mesh=plsc.VectorSubcoreMesh(core_axis_name="c", subcore_axis_name="s"),
        scratch_types=[
            pltpu.VMEM_SHARED((N_NODES, D), jnp.float32),
            pltpu.VMEM((CHUNK,), jnp.int32),
            pltpu.VMEM((CHUNK,), jnp.int32),
            pltpu.VMEM((CHUNK,), jnp.int32),
            pltpu.VMEM((CHUNK,), jnp.int32),
            pltpu.VMEM((CHUNK, D), jnp.float32),
            pltpu.VMEM((CHUNK, D), jnp.float32),
            pltpu.SemaphoreType.DMA,
            pltpu.SemaphoreType.DMA,
            pltpu.SemaphoreType.DMA,
            pltpu.SemaphoreType.DMA,
        ],
    )


def kernel(user_features, item_features, edge_index, Wu, bu, Wi, bi,
           Wum, bum, Wim, bim):
    uemb, gu, iemb, gi = _tc_stage(user_features, item_features,
                                   Wu, bu, Wi, bi, Wum, bum, Wim, bim)
    u_idx = edge_index[0].astype(jnp.int32)
    i_idx = edge_index[1].astype(jnp.int32)
    user_out, item_out = _sc_stage()(gu, gi, uemb, iemb, u_idx, i_idx)
    return (user_out, item_out)
